# baseline (speedup 1.0000x reference)
import jax
import jax.numpy as jnp
from jax.experimental import pallas as pl

HEADS = 16
BUCKETS = 64
DIM = 64
SEQ = 8192
ROWS_PER_BUCKET = SEQ // BUCKETS


def _body(q_ref, k_ref, pq_ref, pk_ref, out_ref):
    qb = q_ref[0]
    kb = k_ref[0]
    inv = 1.0 / ROWS_PER_BUCKET
    sq = qb.reshape(BUCKETS, ROWS_PER_BUCKET, DIM).sum(axis=1) * inv
    sk = kb.reshape(BUCKETS, ROWS_PER_BUCKET, DIM).sum(axis=1) * inv
    sq = sq + pq_ref[0, 0]
    sk = sk + pk_ref[0, 0]
    r = jax.lax.dot_general(
        sq, sk, (((1,), (1,)), ((), ())), preferred_element_type=jnp.float32
    )
    r = r - jnp.max(r, axis=-1, keepdims=True)
    e = jnp.exp(r)
    out_ref[0] = e / jnp.sum(e, axis=-1, keepdims=True)


def kernel(q, k, q_pos_emb, k_pos_emb):
    bh = q.shape[0]
    return pl.pallas_call(
        _body,
        grid=(bh,),
        in_specs=[
            pl.BlockSpec((1, SEQ, DIM), lambda i: (i, 0, 0)),
            pl.BlockSpec((1, SEQ, DIM), lambda i: (i, 0, 0)),
            pl.BlockSpec((1, 1, BUCKETS, DIM), lambda i: (0, i % HEADS, 0, 0)),
            pl.BlockSpec((1, 1, BUCKETS, DIM), lambda i: (0, i % HEADS, 0, 0)),
        ],
        out_specs=pl.BlockSpec((1, BUCKETS, BUCKETS), lambda i: (i, 0, 0)),
        out_shape=jax.ShapeDtypeStruct((bh, BUCKETS, BUCKETS), jnp.float32),
    )(q, k, q_pos_emb, k_pos_emb)
